# Initial kernel scaffold; baseline (speedup 1.0000x reference)
import jax
import jax.numpy as jnp
from jax.experimental import pallas as pl


def kernel(x, edge_index, W_gnn, Ws1, bs1, Ws2, bs2, Wd1, bd1, g1, be1, Wd2, bd2, g2, be2, Wd3, bd3):
    raise NotImplementedError("write your pallas kernel here")


# speedup vs baseline: 3.3123x; 3.3123x over previous
import functools

import jax
import jax.numpy as jnp
from jax import lax
from jax.experimental import pallas as pl
from jax.experimental.pallas import tpu as pltpu
from jax.experimental.pallas import tpu_sc as plsc

N = 10000
E = 320000
D = 128

NC = 2
NS = 16
NW = NC * NS
CHUNK = 128
BLK = 16
NBLK = 5
CHUNKS_PER_W = NBLK * BLK
E_PAD = NW * CHUNKS_PER_W * CHUNK
ACC_ROWS = 10240
ZROWS = 640


def _sc_segment_sum(src_hbm, dst_hbm, x_hbm, zeros_hbm, out_hbm,
                    srcbuf, dstbuf, rows, accum, sem_a, sem_b, sem_i):
  cid = lax.axis_index("c")
  sid = lax.axis_index("s")

  pltpu.sync_copy(zeros_hbm, accum.at[pl.ds(sid * ZROWS, ZROWS)])

  wid = sid * NC + cid

  pltpu.sync_copy(src_hbm.at[wid, 0], srcbuf.at[0])
  pltpu.sync_copy(dst_hbm.at[wid, 0], dstbuf.at[0])
  plsc.subcore_barrier()

  pltpu.async_copy(x_hbm.at[srcbuf.at[0, 0]], rows.at[0], sem_a)
  pltpu.async_copy(x_hbm.at[srcbuf.at[0, 1]], rows.at[1], sem_b)

  def block(b, carry):
    bs = lax.rem(b, 2)
    nbs = lax.rem(b + 1, 2)
    not_last = b + 1 < NBLK

    @pl.when(not_last)
    def _():
      pltpu.async_copy(src_hbm.at[wid, b + 1], srcbuf.at[nbs], sem_i)
      pltpu.async_copy(dst_hbm.at[wid, b + 1], dstbuf.at[nbs], sem_i)

    for p in range(BLK // 2):
      k0, k1 = 2 * p, 2 * p + 1
      if p == BLK // 2 - 1:
        @pl.when(not_last)
        def _():
          pltpu.make_async_copy(src_hbm.at[wid, b + 1], srcbuf.at[nbs],
                                sem_i).wait()
          pltpu.make_async_copy(dst_hbm.at[wid, b + 1], dstbuf.at[nbs],
                                sem_i).wait()

      pltpu.make_async_copy(x_hbm.at[srcbuf.at[bs, k0]], rows.at[0],
                            sem_a).wait()
      pltpu.sync_copy(rows.at[0], accum.at[dstbuf.at[bs, k0]], add=True)
      if p == BLK // 2 - 1:
        @pl.when(not_last)
        def _():
          pltpu.async_copy(x_hbm.at[srcbuf.at[nbs, 0]], rows.at[0], sem_a)
      else:
        pltpu.async_copy(x_hbm.at[srcbuf.at[bs, k0 + 2]], rows.at[0], sem_a)

      pltpu.make_async_copy(x_hbm.at[srcbuf.at[bs, k1]], rows.at[1],
                            sem_b).wait()
      pltpu.sync_copy(rows.at[1], accum.at[dstbuf.at[bs, k1]], add=True)
      if p == BLK // 2 - 1:
        @pl.when(not_last)
        def _():
          pltpu.async_copy(x_hbm.at[srcbuf.at[nbs, 1]], rows.at[1], sem_b)
      else:
        pltpu.async_copy(x_hbm.at[srcbuf.at[bs, k1 + 2]], rows.at[1], sem_b)

    return carry

  lax.fori_loop(0, NBLK, block, None)
  plsc.subcore_barrier()

  @pl.when(sid < 15)
  def _():
    pltpu.sync_copy(accum.at[pl.ds(sid * 624, 624)],
                    out_hbm.at[cid, pl.ds(sid * 624, 624)])

  @pl.when(sid == 15)
  def _():
    pltpu.sync_copy(accum.at[pl.ds(9360, 640)],
                    out_hbm.at[cid, pl.ds(9360, 640)])


_sc_seg_sum_call = functools.partial(
    pl.kernel,
    out_type=jax.ShapeDtypeStruct((NC, N, D), jnp.float32),
    mesh=plsc.VectorSubcoreMesh(core_axis_name="c", subcore_axis_name="s"),
    scratch_types=[
        pltpu.VMEM((2, BLK, CHUNK), jnp.int32),
        pltpu.VMEM((2, BLK, CHUNK), jnp.int32),
        pltpu.VMEM((2, CHUNK, D), jnp.float32),
        pltpu.VMEM_SHARED((ACC_ROWS, D), jnp.float32),
        pltpu.SemaphoreType.DMA,
        pltpu.SemaphoreType.DMA,
        pltpu.SemaphoreType.DMA,
    ],
)(_sc_segment_sum)


def _tc_rest(p_ref, wg_ref, ws1_ref, bs1_ref, ws2_ref, bs2_ref,
             wd1_ref, bd1_ref, g1_ref, be1_ref,
             wd2_ref, bd2_ref, g2_ref, be2_ref,
             wd3_ref, bd3_ref, out_ref):
  aggx = p_ref[0] + p_ref[1]
  agg = jnp.dot(aggx, wg_ref[...], preferred_element_type=jnp.float32,
                precision=jax.lax.Precision.HIGHEST)
  emb = jnp.maximum(agg, 0.0)

  summary = jnp.mean(emb, axis=0, keepdims=True)
  s1 = jnp.maximum(
      jnp.dot(summary, ws1_ref[...], preferred_element_type=jnp.float32)
      + bs1_ref[...], 0.0)
  summary = (jnp.dot(s1, ws2_ref[...], preferred_element_type=jnp.float32)
             + bs2_ref[...])

  z = emb * summary

  h1 = jnp.dot(z, wd1_ref[...], preferred_element_type=jnp.float32) + bd1_ref[...]
  mu1 = jnp.mean(h1, axis=0, keepdims=True)
  var1 = jnp.mean((h1 - mu1) ** 2, axis=0, keepdims=True)
  h1 = g1_ref[...] * (h1 - mu1) * jax.lax.rsqrt(var1 + 1e-5) + be1_ref[...]
  h1 = jnp.maximum(h1, 0.0)

  h2 = jnp.dot(h1, wd2_ref[...], preferred_element_type=jnp.float32) + bd2_ref[...]
  mu2 = jnp.mean(h2, axis=0, keepdims=True)
  var2 = jnp.mean((h2 - mu2) ** 2, axis=0, keepdims=True)
  h2 = g2_ref[...] * (h2 - mu2) * jax.lax.rsqrt(var2 + 1e-5) + be2_ref[...]
  h2 = jnp.maximum(h2, 0.0)

  out_ref[...] = (jnp.dot(h2, wd3_ref[...], preferred_element_type=jnp.float32)
                  + bd3_ref[...])


def kernel(x, edge_index, W_gnn, Ws1, bs1, Ws2, bs2, Wd1, bd1, g1, be1,
           Wd2, bd2, g2, be2, Wd3, bd3):
  src = edge_index[0]
  dst = edge_index[1]
  pad = E_PAD - E
  src_p = jnp.concatenate(
      [src, jnp.zeros((pad,), jnp.int32)]).reshape(NW, NBLK, BLK, CHUNK)
  dst_p = jnp.concatenate(
      [dst, jnp.full((pad,), N, jnp.int32)]).reshape(NW, NBLK, BLK, CHUNK)
  zeros = jnp.zeros((ZROWS, D), jnp.float32)

  partial = _sc_seg_sum_call(src_p, dst_p, x, zeros)

  out = pl.pallas_call(
      _tc_rest,
      out_shape=jax.ShapeDtypeStruct((N, 1), jnp.float32),
  )(partial, W_gnn, Ws1, bs1.reshape(1, D), Ws2, bs2.reshape(1, D),
    Wd1, bd1.reshape(1, 2 * D), g1.reshape(1, 2 * D), be1.reshape(1, 2 * D),
    Wd2, bd2.reshape(1, D), g2.reshape(1, D), be2.reshape(1, D),
    Wd3, bd3.reshape(1, 1))
  return out[:, 0]


# speedup vs baseline: 3.9803x; 1.2017x over previous
import functools

import jax
import jax.numpy as jnp
from jax import lax
from jax.experimental import pallas as pl
from jax.experimental.pallas import tpu as pltpu
from jax.experimental.pallas import tpu_sc as plsc

N = 10000
E = 320000
D = 128

NC = 2
NS = 16
NW = NC * NS
CHUNK = 128
BLK = 16
NBLK_PAIR = 10
NBLK0 = 8
NBLK1 = NBLK_PAIR - NBLK0
E_PAD = NS * NBLK_PAIR * BLK * CHUNK
ACC_ROWS = 10240
ZROWS = 640


def _sc_segment_sum(src_hbm, dst_hbm, x_hbm, zeros_hbm, out_hbm,
                    srcbuf, dstbuf, rows, accum, sem_a, sem_b, sem_i):
  cid = lax.axis_index("c")
  sid = lax.axis_index("s")

  pltpu.sync_copy(zeros_hbm, accum.at[pl.ds(sid * ZROWS, ZROWS)])

  blk0 = cid * NBLK0
  nblk = jnp.where(cid == 0, NBLK0, NBLK1)

  pltpu.sync_copy(src_hbm.at[sid, blk0], srcbuf.at[0])
  pltpu.sync_copy(dst_hbm.at[sid, blk0], dstbuf.at[0])
  plsc.subcore_barrier()

  pltpu.async_copy(x_hbm.at[srcbuf.at[0, 0]], rows.at[0], sem_a)
  pltpu.async_copy(x_hbm.at[srcbuf.at[0, 1]], rows.at[1], sem_b)

  def block(b, carry):
    bs = lax.rem(b, 2)
    nbs = lax.rem(b + 1, 2)
    not_last = b + 1 < nblk

    @pl.when(not_last)
    def _():
      pltpu.async_copy(src_hbm.at[sid, blk0 + b + 1], srcbuf.at[nbs], sem_i)
      pltpu.async_copy(dst_hbm.at[sid, blk0 + b + 1], dstbuf.at[nbs], sem_i)

    for p in range(BLK // 2):
      k0, k1 = 2 * p, 2 * p + 1
      if p == BLK // 2 - 1:
        @pl.when(not_last)
        def _():
          pltpu.make_async_copy(src_hbm.at[sid, blk0 + b + 1],
                                srcbuf.at[nbs], sem_i).wait()
          pltpu.make_async_copy(dst_hbm.at[sid, blk0 + b + 1],
                                dstbuf.at[nbs], sem_i).wait()

      pltpu.make_async_copy(x_hbm.at[srcbuf.at[bs, k0]], rows.at[0],
                            sem_a).wait()
      pltpu.sync_copy(rows.at[0], accum.at[dstbuf.at[bs, k0]], add=True)
      if p == BLK // 2 - 1:
        @pl.when(not_last)
        def _():
          pltpu.async_copy(x_hbm.at[srcbuf.at[nbs, 0]], rows.at[0], sem_a)
      else:
        pltpu.async_copy(x_hbm.at[srcbuf.at[bs, k0 + 2]], rows.at[0], sem_a)

      pltpu.make_async_copy(x_hbm.at[srcbuf.at[bs, k1]], rows.at[1],
                            sem_b).wait()
      pltpu.sync_copy(rows.at[1], accum.at[dstbuf.at[bs, k1]], add=True)
      if p == BLK // 2 - 1:
        @pl.when(not_last)
        def _():
          pltpu.async_copy(x_hbm.at[srcbuf.at[nbs, 1]], rows.at[1], sem_b)
      else:
        pltpu.async_copy(x_hbm.at[srcbuf.at[bs, k1 + 2]], rows.at[1], sem_b)

    return carry

  lax.fori_loop(0, nblk, block, None)
  plsc.subcore_barrier()

  @pl.when(sid < 15)
  def _():
    pltpu.sync_copy(accum.at[pl.ds(sid * 624, 624)],
                    out_hbm.at[cid, pl.ds(sid * 624, 624)])

  @pl.when(sid == 15)
  def _():
    pltpu.sync_copy(accum.at[pl.ds(9360, 640)],
                    out_hbm.at[cid, pl.ds(9360, 640)])


_sc_seg_sum_call = functools.partial(
    pl.kernel,
    out_type=jax.ShapeDtypeStruct((NC, N, D), jnp.float32),
    mesh=plsc.VectorSubcoreMesh(core_axis_name="c", subcore_axis_name="s"),
    scratch_types=[
        pltpu.VMEM((2, BLK, CHUNK), jnp.int32),
        pltpu.VMEM((2, BLK, CHUNK), jnp.int32),
        pltpu.VMEM((2, CHUNK, D), jnp.float32),
        pltpu.VMEM_SHARED((ACC_ROWS, D), jnp.float32),
        pltpu.SemaphoreType.DMA,
        pltpu.SemaphoreType.DMA,
        pltpu.SemaphoreType.DMA,
    ],
)(_sc_segment_sum)


def _tc_rest(p_ref, wg_ref, ws1_ref, bs1_ref, ws2_ref, bs2_ref,
             wd1_ref, bd1_ref, g1_ref, be1_ref,
             wd2_ref, bd2_ref, g2_ref, be2_ref,
             wd3_ref, bd3_ref, out_ref):
  aggx = p_ref[0] + p_ref[1]
  agg = jnp.dot(aggx, wg_ref[...], preferred_element_type=jnp.float32,
                precision=jax.lax.Precision.HIGHEST)
  emb = jnp.maximum(agg, 0.0)

  summary = jnp.mean(emb, axis=0, keepdims=True)
  s1 = jnp.maximum(
      jnp.dot(summary, ws1_ref[...], preferred_element_type=jnp.float32)
      + bs1_ref[...], 0.0)
  summary = (jnp.dot(s1, ws2_ref[...], preferred_element_type=jnp.float32)
             + bs2_ref[...])

  z = emb * summary

  h1 = jnp.dot(z, wd1_ref[...], preferred_element_type=jnp.float32) + bd1_ref[...]
  mu1 = jnp.mean(h1, axis=0, keepdims=True)
  var1 = jnp.mean((h1 - mu1) ** 2, axis=0, keepdims=True)
  h1 = g1_ref[...] * (h1 - mu1) * jax.lax.rsqrt(var1 + 1e-5) + be1_ref[...]
  h1 = jnp.maximum(h1, 0.0)

  h2 = jnp.dot(h1, wd2_ref[...], preferred_element_type=jnp.float32) + bd2_ref[...]
  mu2 = jnp.mean(h2, axis=0, keepdims=True)
  var2 = jnp.mean((h2 - mu2) ** 2, axis=0, keepdims=True)
  h2 = g2_ref[...] * (h2 - mu2) * jax.lax.rsqrt(var2 + 1e-5) + be2_ref[...]
  h2 = jnp.maximum(h2, 0.0)

  out_ref[...] = (jnp.dot(h2, wd3_ref[...], preferred_element_type=jnp.float32)
                  + bd3_ref[...])


def kernel(x, edge_index, W_gnn, Ws1, bs1, Ws2, bs2, Wd1, bd1, g1, be1,
           Wd2, bd2, g2, be2, Wd3, bd3):
  src = edge_index[0]
  dst = edge_index[1]
  pad = E_PAD - E
  src_p = jnp.concatenate(
      [src, jnp.zeros((pad,), jnp.int32)]).reshape(NS, NBLK_PAIR, BLK, CHUNK)
  dst_p = jnp.concatenate(
      [dst, jnp.full((pad,), N, jnp.int32)]).reshape(NS, NBLK_PAIR, BLK, CHUNK)
  zeros = jnp.zeros((ZROWS, D), jnp.float32)

  partial = _sc_seg_sum_call(src_p, dst_p, x, zeros)

  out = pl.pallas_call(
      _tc_rest,
      out_shape=jax.ShapeDtypeStruct((N, 1), jnp.float32),
  )(partial, W_gnn, Ws1, bs1.reshape(1, D), Ws2, bs2.reshape(1, D),
    Wd1, bd1.reshape(1, 2 * D), g1.reshape(1, 2 * D), be1.reshape(1, 2 * D),
    Wd2, bd2.reshape(1, D), g2.reshape(1, D), be2.reshape(1, D),
    Wd3, bd3.reshape(1, 1))
  return out[:, 0]
